# Initial kernel scaffold; baseline (speedup 1.0000x reference)
import jax
import jax.numpy as jnp
from jax.experimental import pallas as pl


def kernel(x_user, x_movie, edge_index, W_user, b_user, W_movie, b_movie, W_conv0, b_conv0, W_conv1, b_conv1, W_edge, b_edge):
    raise NotImplementedError("write your pallas kernel here")


# speedup vs baseline: 11.3777x; 11.3777x over previous
import functools

import jax
import jax.numpy as jnp
from jax import lax
from jax.experimental import pallas as pl
from jax.experimental.pallas import tpu as pltpu
from jax.experimental.pallas import tpu_sc as plsc

Nu, Nm, E, D, H, C = 5000, 5000, 320000, 128, 128, 2
MOFF = 5120
NP = 2 * MOFF
NC, NS = 2, 16
NT = NC * NS
EPT = E // NT
KA = 80
NKA = EPT // KA
ROWS_PT = MOFF // NS

_mesh = plsc.VectorSubcoreMesh(core_axis_name="c", subcore_axis_name="s")


def _deg_body(col_hbm, out_hbm, acc_v, cidx_v):
    cid = lax.axis_index("c")
    sid = lax.axis_index("s")
    wid = cid * NS + sid

    @pl.loop(0, MOFF // 16)
    def _zero(i):
        acc_v[pl.ds(i * 16, 16)] = jnp.zeros((16,), jnp.float32)

    base = wid * EPT
    ones = jnp.ones((16,), jnp.float32)

    @pl.loop(0, NKR)
    def _chunks(k):
        pltpu.sync_copy(col_hbm.at[pl.ds(base + k * KR, KR)], cidx_v)

        @pl.loop(0, KR // 16)
        def _steps(j):
            iv = cidx_v[pl.ds(j * 16, 16)]
            plsc.addupdate_scatter(acc_v, [iv], ones)

    pltpu.sync_copy(acc_v, out_hbm.at[wid])


def _agg_body(z_hbm, row_hbm, col_hbm, zeros_hbm, out_hbm,
                ridx_v, cidx_v, rows_v, agg_sh, sem):
    cid = lax.axis_index("c")
    sid = lax.axis_index("s")
    wid = cid * NS + sid
    pltpu.sync_copy(zeros_hbm, agg_sh.at[pl.ds(sid * ROWS_PT, ROWS_PT)])
    plsc.subcore_barrier()
    base = wid * EPT

    @pl.loop(0, NKA)
    def _chunks(k):
        pltpu.sync_copy(row_hbm.at[pl.ds(base + k * KA, KA)], ridx_v)
        pltpu.sync_copy(col_hbm.at[pl.ds(base + k * KA, KA)], cidx_v)
        pltpu.async_copy(z_hbm.at[ridx_v], rows_v, sem).wait()
        pltpu.sync_copy(rows_v, agg_sh.at[cidx_v], add=True)

    plsc.subcore_barrier()
    pltpu.sync_copy(agg_sh.at[pl.ds(sid * ROWS_PT, ROWS_PT)],
                    out_hbm.at[cid, pl.ds(sid * ROWS_PT, ROWS_PT)])


KR = 400
NKR = EPT // KR


def _readout_body(tab_hbm, row_hbm, col_hbm, out_hbm,
                    tab_v, ridx_v, cidx_v, obuf_v):
    cid = lax.axis_index("c")
    sid = lax.axis_index("s")
    wid = cid * NS + sid
    pltpu.sync_copy(tab_hbm, tab_v)
    base = wid * EPT
    ii = lax.iota(jnp.int32, 16)

    @pl.loop(0, NKR)
    def _chunks(k):
        pltpu.sync_copy(row_hbm.at[pl.ds(base + k * KR, KR)], ridx_v)
        pltpu.sync_copy(col_hbm.at[pl.ds(base + k * KR, KR)], cidx_v)

        @pl.loop(0, KR // 16)
        def _steps(j):
            rv = ridx_v[pl.ds(j * 16, 16)]
            cv = cidx_v[pl.ds(j * 16, 16)] + MOFF
            v0 = plsc.load_gather(tab_v, [rv])
            v1 = plsc.load_gather(tab_v, [rv + NP])
            w0 = plsc.load_gather(tab_v, [cv + 2 * NP])
            w1 = plsc.load_gather(tab_v, [cv + 3 * NP])
            at = j * 32 + 2 * ii
            plsc.store_scatter(obuf_v, [at], v0 + w0)
            plsc.store_scatter(obuf_v, [at + 1], v1 + w1)

        pltpu.sync_copy(obuf_v, out_hbm.at[pl.ds(2 * (base + k * KR), 2 * KR)])


_DEG_KW = dict(
    out_type=jax.ShapeDtypeStruct((NT, MOFF), jnp.float32),
    mesh=_mesh,
    scratch_types=[
        pltpu.VMEM((MOFF,), jnp.float32),
        pltpu.VMEM((KR,), jnp.int32),
    ],
    compiler_params=pltpu.CompilerParams(needs_layout_passes=False),
)
_AGG_KW = dict(
    out_type=jax.ShapeDtypeStruct((NC, MOFF, D), jnp.float32),
    mesh=_mesh,
    scratch_types=[
        pltpu.VMEM((KA,), jnp.int32),
        pltpu.VMEM((KA,), jnp.int32),
        pltpu.VMEM((KA, D), jnp.float32),
        pltpu.VMEM_SHARED((MOFF, D), jnp.float32),
        pltpu.SemaphoreType.DMA,
    ],
)
_READOUT_KW = dict(
    out_type=jax.ShapeDtypeStruct((2 * E,), jnp.float32),
    mesh=_mesh,
    scratch_types=[
        pltpu.VMEM((4 * NP,), jnp.float32),
        pltpu.VMEM((KR,), jnp.int32),
        pltpu.VMEM((KR,), jnp.int32),
        pltpu.VMEM((2 * KR,), jnp.float32),
    ],
    compiler_params=pltpu.CompilerParams(needs_layout_passes=False),
)

_deg_kernel = pl.kernel(_deg_body, **_DEG_KW)
_agg_kernel = pl.kernel(_agg_body, **_AGG_KW)
_readout_kernel = pl.kernel(_readout_body, **_READOUT_KW)


BLK = 512
NBLK = NP // BLK
UBLK = MOFF // BLK


def _dis_from_cnt(cnt_blk, b):
    cnt0 = lax.dot_general(cnt_blk, jnp.ones((NT, 1), jnp.float32),
                           (((0,), (0,)), ((), ())),
                           preferred_element_type=jnp.float32)
    cnt0 = jnp.where(b < UBLK, cnt0, 0.0)
    return lax.rsqrt(1.0 + cnt0)


def _mm_nt(a, w):
    return lax.dot_general(a, w, (((1,), (1,)), ((), ())),
                           preferred_element_type=jnp.float32)


def _z1_body(x_ref, wu_ref, bu_ref, wm_ref, bm_ref, wc_ref, bc_ref, cnt_ref,
             z_ref):
    b = pl.program_id(0)
    is_user = b < UBLK
    wp = jnp.where(is_user, wu_ref[...], wm_ref[...])
    bp = jnp.where(is_user, bu_ref[...], bm_ref[...])
    h = _mm_nt(x_ref[...], wp) + bp
    hc = _mm_nt(h, wc_ref[...]) + bc_ref[...]
    z_ref[...] = _dis_from_cnt(cnt_ref[...], b) * hc


def _z2_body(z_ref, p_ref, cnt_ref, wc_ref, bc_ref, z2_ref):
    b = pl.program_id(0)
    edge = jnp.where(b < UBLK, p_ref[0] + p_ref[1], 0.0)
    dis = _dis_from_cnt(cnt_ref[...], b)
    x1 = jnp.maximum(dis * (z_ref[...] + edge), 0.0)
    z2_ref[...] = dis * (_mm_nt(x1, wc_ref[...]) + bc_ref[...])


def _proj_body(z_ref, p_ref, cnt_ref, we_ref, be_ref, t_ref):
    b = pl.program_id(0)
    edge = jnp.where(b < UBLK, p_ref[0] + p_ref[1], 0.0)
    dis = _dis_from_cnt(cnt_ref[...], b)
    x2 = jnp.maximum(dis * (z_ref[...] + edge), 0.0)
    t_ref[...] = lax.dot_general(we_ref[...], x2, (((1,), (1,)), ((), ())),
                                 preferred_element_type=jnp.float32) \
        + be_ref[...][:, 0:1]


def _full(shape):
    return pl.BlockSpec(shape, lambda b: tuple(0 for _ in shape))


_row_spec = pl.BlockSpec((BLK, D), lambda b: (b, 0))
_cnt_spec = pl.BlockSpec((NT, BLK),
                         lambda b: (0, jnp.minimum(b, UBLK - 1)))
_p_spec = pl.BlockSpec((NC, BLK, D),
                       lambda b: (0, jnp.minimum(b, UBLK - 1), 0))


def kernel(x_user, x_movie, edge_index, W_user, b_user, W_movie, b_movie,
           W_conv0, b_conv0, W_conv1, b_conv1, W_edge, b_edge):
    f32 = jnp.float32
    row = edge_index[0]
    col = edge_index[1]
    x_p = jnp.zeros((NP, D), f32)
    x_p = x_p.at[:Nu].set(x_user).at[MOFF:MOFF + Nm].set(x_movie)

    zerosD = jnp.zeros((ROWS_PT, D), f32)

    cnt = _deg_kernel(col)

    z1 = pl.pallas_call(
        _z1_body,
        grid=(NBLK,),
        in_specs=[_row_spec, _full((D, D)), _full((1, D)), _full((D, D)),
                  _full((1, D)), _full((D, D)), _full((1, D)), _cnt_spec],
        out_specs=_row_spec,
        out_shape=jax.ShapeDtypeStruct((NP, D), f32),
    )(x_p, W_user, b_user.reshape(1, D), W_movie, b_movie.reshape(1, D),
      W_conv0, b_conv0.reshape(1, D), cnt)

    p1 = _agg_kernel(z1, row, col, zerosD)

    z2 = pl.pallas_call(
        _z2_body,
        grid=(NBLK,),
        in_specs=[_row_spec, _p_spec, _cnt_spec, _full((D, D)), _full((1, D))],
        out_specs=_row_spec,
        out_shape=jax.ShapeDtypeStruct((NP, D), f32),
    )(z1, p1, cnt, W_conv1, b_conv1.reshape(1, D))

    p2 = _agg_kernel(z2, row, col, zerosD)

    w_comb = jnp.zeros((8, D), f32)
    w_comb = w_comb.at[0:2].set(W_edge[:, :H]).at[2:4].set(W_edge[:, H:])
    b_comb = jnp.zeros((8, D), f32)
    b_comb = b_comb.at[0, :].set(b_edge[0]).at[1, :].set(b_edge[1])

    tab = pl.pallas_call(
        _proj_body,
        grid=(NBLK,),
        in_specs=[_row_spec, _p_spec, _cnt_spec, _full((8, D)), _full((8, D))],
        out_specs=pl.BlockSpec((8, BLK), lambda b: (0, b)),
        out_shape=jax.ShapeDtypeStruct((8, NP), f32),
    )(z2, p2, cnt, w_comb, b_comb)

    out = _readout_kernel(tab[:4].reshape(4 * NP), row, col)
    return out.reshape(E, C)


# speedup vs baseline: 19.0047x; 1.6703x over previous
import functools

import jax
import jax.numpy as jnp
from jax import lax
from jax.experimental import pallas as pl
from jax.experimental.pallas import tpu as pltpu
from jax.experimental.pallas import tpu_sc as plsc

Nu, Nm, E, D, H, C = 5000, 5000, 320000, 128, 128, 2
MOFF = 5120
NP = 2 * MOFF
NC, NS = 2, 16
NT = NC * NS
EPT = E // NT
KA = 200
NKA = EPT // KA
NBUF = 2
ROWS_PT = MOFF // NS

_mesh = plsc.VectorSubcoreMesh(core_axis_name="c", subcore_axis_name="s")


def _deg_body(col_hbm, out_hbm, acc_v, cidx_v):
    cid = lax.axis_index("c")
    sid = lax.axis_index("s")
    wid = cid * NS + sid

    @pl.loop(0, MOFF // 16)
    def _zero(i):
        acc_v[pl.ds(i * 16, 16)] = jnp.zeros((16,), jnp.float32)

    base = wid * EPT
    ones = jnp.ones((16,), jnp.float32)

    @pl.loop(0, NKR)
    def _chunks(k):
        pltpu.sync_copy(col_hbm.at[pl.ds(base + k * KR, KR)], cidx_v)

        @pl.loop(0, KR // 16)
        def _steps(j):
            iv = cidx_v[pl.ds(j * 16, 16)]
            plsc.addupdate_scatter(acc_v, [iv], ones)

    pltpu.sync_copy(acc_v, out_hbm.at[wid])


def _agg_body(z_hbm, row_hbm, col_hbm, zeros_hbm, out_hbm,
                ridx_v, cidx0_v, cidx1_v, rows0_v, rows1_v, agg_sh,
                gsem0, gsem1, isem0, isem1):
    cid = lax.axis_index("c")
    sid = lax.axis_index("s")
    wid = cid * NS + sid
    pltpu.sync_copy(zeros_hbm, agg_sh.at[pl.ds(sid * ROWS_PT, ROWS_PT)])
    base = wid * EPT
    pltpu.sync_copy(row_hbm.at[pl.ds(base, EPT)], ridx_v)
    plsc.subcore_barrier()

    cbufs = (cidx0_v, cidx1_v)
    rbufs = (rows0_v, rows1_v)
    gsems = (gsem0, gsem1)
    isems = (isem0, isem1)

    def _issue(k, b):
        pltpu.async_copy(col_hbm.at[pl.ds(base + k * KA, KA)],
                         cbufs[b], isems[b])
        pltpu.async_copy(z_hbm.at[ridx_v.at[pl.ds(k * KA, KA)]],
                         rbufs[b], gsems[b])

    for b in range(NBUF):
        _issue(b, b)

    @pl.loop(0, NKA // NBUF)
    def _groups(g):
        for b in range(NBUF):
            k = g * NBUF + b
            pltpu.make_async_copy(col_hbm.at[pl.ds(base + k * KA, KA)],
                                  cbufs[b], isems[b]).wait()
            pltpu.make_async_copy(z_hbm.at[ridx_v.at[pl.ds(k * KA, KA)]],
                                  rbufs[b], gsems[b]).wait()
            pltpu.sync_copy(rbufs[b], agg_sh.at[cbufs[b]], add=True)

            @pl.when(k + NBUF < NKA)
            def _():
                _issue(k + NBUF, b)

    plsc.subcore_barrier()
    pltpu.sync_copy(agg_sh.at[pl.ds(sid * ROWS_PT, ROWS_PT)],
                    out_hbm.at[cid, pl.ds(sid * ROWS_PT, ROWS_PT)])


KR = 400
NKR = EPT // KR


def _readout_body(tab_hbm, row_hbm, col_hbm, out_hbm,
                    tab_v, ridx_v, cidx_v, obuf_v):
    cid = lax.axis_index("c")
    sid = lax.axis_index("s")
    wid = cid * NS + sid
    pltpu.sync_copy(tab_hbm, tab_v)
    base = wid * EPT
    ii = lax.iota(jnp.int32, 16)

    @pl.loop(0, NKR)
    def _chunks(k):
        pltpu.sync_copy(row_hbm.at[pl.ds(base + k * KR, KR)], ridx_v)
        pltpu.sync_copy(col_hbm.at[pl.ds(base + k * KR, KR)], cidx_v)

        @pl.loop(0, KR // 16)
        def _steps(j):
            rv = ridx_v[pl.ds(j * 16, 16)]
            cv = cidx_v[pl.ds(j * 16, 16)] + MOFF
            v0 = plsc.load_gather(tab_v, [rv])
            v1 = plsc.load_gather(tab_v, [rv + NP])
            w0 = plsc.load_gather(tab_v, [cv + 2 * NP])
            w1 = plsc.load_gather(tab_v, [cv + 3 * NP])
            at = j * 32 + 2 * ii
            plsc.store_scatter(obuf_v, [at], v0 + w0)
            plsc.store_scatter(obuf_v, [at + 1], v1 + w1)

        pltpu.sync_copy(obuf_v, out_hbm.at[pl.ds(2 * (base + k * KR), 2 * KR)])


_DEG_KW = dict(
    out_type=jax.ShapeDtypeStruct((NT, MOFF), jnp.float32),
    mesh=_mesh,
    scratch_types=[
        pltpu.VMEM((MOFF,), jnp.float32),
        pltpu.VMEM((KR,), jnp.int32),
    ],
    compiler_params=pltpu.CompilerParams(needs_layout_passes=False),
)
_AGG_KW = dict(
    out_type=jax.ShapeDtypeStruct((NC, MOFF, D), jnp.float32),
    mesh=_mesh,
    scratch_types=[
        pltpu.VMEM((EPT,), jnp.int32),
        pltpu.VMEM((KA,), jnp.int32),
        pltpu.VMEM((KA,), jnp.int32),
        pltpu.VMEM((KA, D), jnp.float32),
        pltpu.VMEM((KA, D), jnp.float32),
        pltpu.VMEM_SHARED((MOFF, D), jnp.float32),
        pltpu.SemaphoreType.DMA,
        pltpu.SemaphoreType.DMA,
        pltpu.SemaphoreType.DMA,
        pltpu.SemaphoreType.DMA,
    ],
)
_READOUT_KW = dict(
    out_type=jax.ShapeDtypeStruct((2 * E,), jnp.float32),
    mesh=_mesh,
    scratch_types=[
        pltpu.VMEM((4 * NP,), jnp.float32),
        pltpu.VMEM((KR,), jnp.int32),
        pltpu.VMEM((KR,), jnp.int32),
        pltpu.VMEM((2 * KR,), jnp.float32),
    ],
    compiler_params=pltpu.CompilerParams(needs_layout_passes=False),
)

_deg_kernel = pl.kernel(_deg_body, **_DEG_KW)
_agg_kernel = pl.kernel(_agg_body, **_AGG_KW)
_readout_kernel = pl.kernel(_readout_body, **_READOUT_KW)


BLK = 512
NBLK = NP // BLK
UBLK = MOFF // BLK


def _dis_from_cnt(cnt_blk, b):
    cnt0 = lax.dot_general(cnt_blk, jnp.ones((NT, 1), jnp.float32),
                           (((0,), (0,)), ((), ())),
                           preferred_element_type=jnp.float32)
    cnt0 = jnp.where(b < UBLK, cnt0, 0.0)
    return lax.rsqrt(1.0 + cnt0)


def _mm_nt(a, w):
    return lax.dot_general(a, w, (((1,), (1,)), ((), ())),
                           preferred_element_type=jnp.float32)


def _z1_body(x_ref, wu_ref, bu_ref, wm_ref, bm_ref, wc_ref, bc_ref, cnt_ref,
             z_ref):
    b = pl.program_id(0)
    is_user = b < UBLK
    wp = jnp.where(is_user, wu_ref[...], wm_ref[...])
    bp = jnp.where(is_user, bu_ref[...], bm_ref[...])
    h = _mm_nt(x_ref[...], wp) + bp
    hc = _mm_nt(h, wc_ref[...]) + bc_ref[...]
    z_ref[...] = _dis_from_cnt(cnt_ref[...], b) * hc


def _z2_body(z_ref, p_ref, cnt_ref, wc_ref, bc_ref, z2_ref):
    b = pl.program_id(0)
    edge = jnp.where(b < UBLK, p_ref[0] + p_ref[1], 0.0)
    dis = _dis_from_cnt(cnt_ref[...], b)
    x1 = jnp.maximum(dis * (z_ref[...] + edge), 0.0)
    z2_ref[...] = dis * (_mm_nt(x1, wc_ref[...]) + bc_ref[...])


def _proj_body(z_ref, p_ref, cnt_ref, we_ref, be_ref, t_ref):
    b = pl.program_id(0)
    edge = jnp.where(b < UBLK, p_ref[0] + p_ref[1], 0.0)
    dis = _dis_from_cnt(cnt_ref[...], b)
    x2 = jnp.maximum(dis * (z_ref[...] + edge), 0.0)
    t_ref[...] = lax.dot_general(we_ref[...], x2, (((1,), (1,)), ((), ())),
                                 preferred_element_type=jnp.float32) \
        + be_ref[...][:, 0:1]


def _full(shape):
    return pl.BlockSpec(shape, lambda b: tuple(0 for _ in shape))


_row_spec = pl.BlockSpec((BLK, D), lambda b: (b, 0))
_cnt_spec = pl.BlockSpec((NT, BLK),
                         lambda b: (0, jnp.minimum(b, UBLK - 1)))
_p_spec = pl.BlockSpec((NC, BLK, D),
                       lambda b: (0, jnp.minimum(b, UBLK - 1), 0))


def kernel(x_user, x_movie, edge_index, W_user, b_user, W_movie, b_movie,
           W_conv0, b_conv0, W_conv1, b_conv1, W_edge, b_edge):
    f32 = jnp.float32
    row = edge_index[0]
    col = edge_index[1]
    x_p = jnp.zeros((NP, D), f32)
    x_p = x_p.at[:Nu].set(x_user).at[MOFF:MOFF + Nm].set(x_movie)

    zerosD = jnp.zeros((ROWS_PT, D), f32)

    cnt = _deg_kernel(col)

    z1 = pl.pallas_call(
        _z1_body,
        grid=(NBLK,),
        in_specs=[_row_spec, _full((D, D)), _full((1, D)), _full((D, D)),
                  _full((1, D)), _full((D, D)), _full((1, D)), _cnt_spec],
        out_specs=_row_spec,
        out_shape=jax.ShapeDtypeStruct((NP, D), f32),
    )(x_p, W_user, b_user.reshape(1, D), W_movie, b_movie.reshape(1, D),
      W_conv0, b_conv0.reshape(1, D), cnt)

    p1 = _agg_kernel(z1, row, col, zerosD)

    z2 = pl.pallas_call(
        _z2_body,
        grid=(NBLK,),
        in_specs=[_row_spec, _p_spec, _cnt_spec, _full((D, D)), _full((1, D))],
        out_specs=_row_spec,
        out_shape=jax.ShapeDtypeStruct((NP, D), f32),
    )(z1, p1, cnt, W_conv1, b_conv1.reshape(1, D))

    p2 = _agg_kernel(z2, row, col, zerosD)

    w_comb = jnp.zeros((8, D), f32)
    w_comb = w_comb.at[0:2].set(W_edge[:, :H]).at[2:4].set(W_edge[:, H:])
    b_comb = jnp.zeros((8, D), f32)
    b_comb = b_comb.at[0, :].set(b_edge[0]).at[1, :].set(b_edge[1])

    tab = pl.pallas_call(
        _proj_body,
        grid=(NBLK,),
        in_specs=[_row_spec, _p_spec, _cnt_spec, _full((8, D)), _full((8, D))],
        out_specs=pl.BlockSpec((8, BLK), lambda b: (0, b)),
        out_shape=jax.ShapeDtypeStruct((8, NP), f32),
    )(z2, p2, cnt, w_comb, b_comb)

    out = _readout_kernel(tab[:4].reshape(4 * NP), row, col)
    return out.reshape(E, C)


# speedup vs baseline: 32.7067x; 1.7210x over previous
import functools

import jax
import jax.numpy as jnp
from jax import lax
from jax.experimental import pallas as pl
from jax.experimental.pallas import tpu as pltpu
from jax.experimental.pallas import tpu_sc as plsc

Nu, Nm, E, D, H, C = 5000, 5000, 320000, 128, 128, 2
MOFF = 5120
NP = 2 * MOFF
NC, NS = 2, 16
NT = NC * NS
EPT = E // NT
KA = 200
NKA = EPT // KA
NBUF = 2
ROWS_PT = MOFF // NS

_mesh = plsc.VectorSubcoreMesh(core_axis_name="c", subcore_axis_name="s")


def _deg_body(col_hbm, out_hbm, acc_v, cidx_v):
    cid = lax.axis_index("c")
    sid = lax.axis_index("s")
    wid = cid * NS + sid

    @pl.loop(0, MOFF // 16)
    def _zero(i):
        acc_v[pl.ds(i * 16, 16)] = jnp.zeros((16,), jnp.float32)

    base = wid * EPT
    ones = jnp.ones((16,), jnp.float32)

    @pl.loop(0, NKR)
    def _chunks(k):
        pltpu.sync_copy(col_hbm.at[pl.ds(base + k * KR, KR)], cidx_v)

        @pl.loop(0, KR // 16)
        def _steps(j):
            iv = cidx_v[pl.ds(j * 16, 16)]
            plsc.addupdate_scatter(acc_v, [iv], ones)

    pltpu.sync_copy(acc_v, out_hbm.at[wid])


def _agg_body(z_hbm, row_hbm, col_hbm, zeros_hbm, out_hbm,
                ridx_v, cidx0_v, cidx1_v, rows0_v, rows1_v, agg_sh,
                gsem0, gsem1, isem0, isem1):
    cid = lax.axis_index("c")
    sid = lax.axis_index("s")
    wid = cid * NS + sid
    pltpu.sync_copy(zeros_hbm, agg_sh.at[pl.ds(sid * ROWS_PT, ROWS_PT)])
    base = wid * EPT
    pltpu.sync_copy(row_hbm.at[pl.ds(base, EPT)], ridx_v)
    plsc.subcore_barrier()

    cbufs = (cidx0_v, cidx1_v)
    rbufs = (rows0_v, rows1_v)
    gsems = (gsem0, gsem1)
    isems = (isem0, isem1)

    def _issue(k, b):
        pltpu.async_copy(col_hbm.at[pl.ds(base + k * KA, KA)],
                         cbufs[b], isems[b])
        pltpu.async_copy(z_hbm.at[ridx_v.at[pl.ds(k * KA, KA)]],
                         rbufs[b], gsems[b])

    for b in range(NBUF):
        _issue(b, b)

    @pl.loop(0, NKA // NBUF)
    def _groups(g):
        for b in range(NBUF):
            k = g * NBUF + b
            pltpu.make_async_copy(col_hbm.at[pl.ds(base + k * KA, KA)],
                                  cbufs[b], isems[b]).wait()
            pltpu.make_async_copy(z_hbm.at[ridx_v.at[pl.ds(k * KA, KA)]],
                                  rbufs[b], gsems[b]).wait()
            pltpu.sync_copy(rbufs[b], agg_sh.at[cbufs[b]], add=True)

            @pl.when(k + NBUF < NKA)
            def _():
                _issue(k + NBUF, b)

    plsc.subcore_barrier()
    pltpu.sync_copy(agg_sh.at[pl.ds(sid * ROWS_PT, ROWS_PT)],
                    out_hbm.at[cid, pl.ds(sid * ROWS_PT, ROWS_PT)])


KR = 400
NKR = EPT // KR

GRP = 128
CGR = 4
CED = CGR * GRP
NCH = E // CED
ITR = (NCH + NT - 1) // NT


def _readout_body(tab_hbm, row_hbm, col_hbm, out_hbm,
                    tab_v, ridx_v, cidx_v, obuf_v):
    cid = lax.axis_index("c")
    sid = lax.axis_index("s")
    wid = cid * NS + sid
    pltpu.sync_copy(tab_hbm, tab_v)

    @pl.loop(0, ITR)
    def _iters(t):
        ch = t * NT + wid

        @pl.when(ch < NCH)
        def _():
            eb = ch * CED
            pltpu.sync_copy(row_hbm.at[pl.ds(eb, CED)], ridx_v)
            pltpu.sync_copy(col_hbm.at[pl.ds(eb, CED)], cidx_v)

            @pl.loop(0, CGR)
            def _grp(q):
                @pl.loop(0, GRP // 16)
                def _vec(jj):
                    o = q * GRP + jj * 16
                    rv = ridx_v[pl.ds(o, 16)]
                    cv = cidx_v[pl.ds(o, 16)] + MOFF
                    v0 = plsc.load_gather(tab_v, [rv]) \
                        + plsc.load_gather(tab_v, [cv + 2 * NP])
                    v1 = plsc.load_gather(tab_v, [rv + NP]) \
                        + plsc.load_gather(tab_v, [cv + 3 * NP])
                    obuf_v[pl.ds(2 * q * GRP + jj * 16, 16)] = v0
                    obuf_v[pl.ds(2 * q * GRP + GRP + jj * 16, 16)] = v1

            pltpu.sync_copy(obuf_v, out_hbm.at[pl.ds(2 * eb, 2 * CED)])


_DEG_KW = dict(
    out_type=jax.ShapeDtypeStruct((NT, MOFF), jnp.float32),
    mesh=_mesh,
    scratch_types=[
        pltpu.VMEM((MOFF,), jnp.float32),
        pltpu.VMEM((KR,), jnp.int32),
    ],
    compiler_params=pltpu.CompilerParams(needs_layout_passes=False),
)
_AGG_KW = dict(
    out_type=jax.ShapeDtypeStruct((NC, MOFF, D), jnp.float32),
    mesh=_mesh,
    scratch_types=[
        pltpu.VMEM((EPT,), jnp.int32),
        pltpu.VMEM((KA,), jnp.int32),
        pltpu.VMEM((KA,), jnp.int32),
        pltpu.VMEM((KA, D), jnp.float32),
        pltpu.VMEM((KA, D), jnp.float32),
        pltpu.VMEM_SHARED((MOFF, D), jnp.float32),
        pltpu.SemaphoreType.DMA,
        pltpu.SemaphoreType.DMA,
        pltpu.SemaphoreType.DMA,
        pltpu.SemaphoreType.DMA,
    ],
)
_READOUT_KW = dict(
    out_type=jax.ShapeDtypeStruct((2 * E,), jnp.float32),
    mesh=_mesh,
    scratch_types=[
        pltpu.VMEM((4 * NP,), jnp.float32),
        pltpu.VMEM((CED,), jnp.int32),
        pltpu.VMEM((CED,), jnp.int32),
        pltpu.VMEM((2 * CED,), jnp.float32),
    ],
    compiler_params=pltpu.CompilerParams(needs_layout_passes=False),
)

_deg_kernel = pl.kernel(_deg_body, **_DEG_KW)
_agg_kernel = pl.kernel(_agg_body, **_AGG_KW)
_readout_kernel = pl.kernel(_readout_body, **_READOUT_KW)


BLK = 512
NBLK = NP // BLK
UBLK = MOFF // BLK


def _dis_from_cnt(cnt_blk, b):
    cnt0 = lax.dot_general(cnt_blk, jnp.ones((NT, 1), jnp.float32),
                           (((0,), (0,)), ((), ())),
                           preferred_element_type=jnp.float32)
    cnt0 = jnp.where(b < UBLK, cnt0, 0.0)
    return lax.rsqrt(1.0 + cnt0)


def _mm_nt(a, w):
    return lax.dot_general(a, w, (((1,), (1,)), ((), ())),
                           preferred_element_type=jnp.float32)


def _z1_body(x_ref, wu_ref, bu_ref, wm_ref, bm_ref, wc_ref, bc_ref, cnt_ref,
             z_ref):
    b = pl.program_id(0)
    is_user = b < UBLK
    wp = jnp.where(is_user, wu_ref[...], wm_ref[...])
    bp = jnp.where(is_user, bu_ref[...], bm_ref[...])
    h = _mm_nt(x_ref[...], wp) + bp
    hc = _mm_nt(h, wc_ref[...]) + bc_ref[...]
    z_ref[...] = _dis_from_cnt(cnt_ref[...], b) * hc


def _z2_body(z_ref, p_ref, cnt_ref, wc_ref, bc_ref, z2_ref):
    b = pl.program_id(0)
    edge = jnp.where(b < UBLK, p_ref[0] + p_ref[1], 0.0)
    dis = _dis_from_cnt(cnt_ref[...], b)
    x1 = jnp.maximum(dis * (z_ref[...] + edge), 0.0)
    z2_ref[...] = dis * (_mm_nt(x1, wc_ref[...]) + bc_ref[...])


def _proj_body(z_ref, p_ref, cnt_ref, we_ref, be_ref, t_ref):
    b = pl.program_id(0)
    edge = jnp.where(b < UBLK, p_ref[0] + p_ref[1], 0.0)
    dis = _dis_from_cnt(cnt_ref[...], b)
    x2 = jnp.maximum(dis * (z_ref[...] + edge), 0.0)
    t_ref[...] = lax.dot_general(we_ref[...], x2, (((1,), (1,)), ((), ())),
                                 preferred_element_type=jnp.float32) \
        + be_ref[...][:, 0:1]


def _full(shape):
    return pl.BlockSpec(shape, lambda b: tuple(0 for _ in shape))


_row_spec = pl.BlockSpec((BLK, D), lambda b: (b, 0))
_cnt_spec = pl.BlockSpec((NT, BLK),
                         lambda b: (0, jnp.minimum(b, UBLK - 1)))
_p_spec = pl.BlockSpec((NC, BLK, D),
                       lambda b: (0, jnp.minimum(b, UBLK - 1), 0))


def kernel(x_user, x_movie, edge_index, W_user, b_user, W_movie, b_movie,
           W_conv0, b_conv0, W_conv1, b_conv1, W_edge, b_edge):
    f32 = jnp.float32
    row = edge_index[0]
    col = edge_index[1]
    x_p = jnp.zeros((NP, D), f32)
    x_p = x_p.at[:Nu].set(x_user).at[MOFF:MOFF + Nm].set(x_movie)

    zerosD = jnp.zeros((ROWS_PT, D), f32)

    cnt = _deg_kernel(col)

    z1 = pl.pallas_call(
        _z1_body,
        grid=(NBLK,),
        in_specs=[_row_spec, _full((D, D)), _full((1, D)), _full((D, D)),
                  _full((1, D)), _full((D, D)), _full((1, D)), _cnt_spec],
        out_specs=_row_spec,
        out_shape=jax.ShapeDtypeStruct((NP, D), f32),
    )(x_p, W_user, b_user.reshape(1, D), W_movie, b_movie.reshape(1, D),
      W_conv0, b_conv0.reshape(1, D), cnt)

    p1 = _agg_kernel(z1, row, col, zerosD)

    z2 = pl.pallas_call(
        _z2_body,
        grid=(NBLK,),
        in_specs=[_row_spec, _p_spec, _cnt_spec, _full((D, D)), _full((1, D))],
        out_specs=_row_spec,
        out_shape=jax.ShapeDtypeStruct((NP, D), f32),
    )(z1, p1, cnt, W_conv1, b_conv1.reshape(1, D))

    p2 = _agg_kernel(z2, row, col, zerosD)

    w_comb = jnp.zeros((8, D), f32)
    w_comb = w_comb.at[0:2].set(W_edge[:, :H]).at[2:4].set(W_edge[:, H:])
    b_comb = jnp.zeros((8, D), f32)
    b_comb = b_comb.at[0, :].set(b_edge[0]).at[1, :].set(b_edge[1])

    tab = pl.pallas_call(
        _proj_body,
        grid=(NBLK,),
        in_specs=[_row_spec, _p_spec, _cnt_spec, _full((8, D)), _full((8, D))],
        out_specs=pl.BlockSpec((8, BLK), lambda b: (0, b)),
        out_shape=jax.ShapeDtypeStruct((8, NP), f32),
    )(z2, p2, cnt, w_comb, b_comb)

    out = _readout_kernel(tab[:4].reshape(4 * NP), row, col)
    return out.reshape(2 * E // 256, 2, 128).transpose(0, 2, 1).reshape(E, C)
